# baseline (device time: 9257 ns/iter reference)
import jax
import jax.numpy as jnp
from jax import lax
from jax.experimental import pallas as pl
from jax.experimental.pallas import tpu as pltpu

H = 512
N = 256
CH = 64
NCH = H // CH


def kernel(x, dest):
    dest_row = dest.reshape(1, H)

    def body(
        x_ref, dest_ref, out_ref,
        sendbuf, recvbuf, destpeer, sems_d, ssems, rsems,
    ):
        my_x = lax.axis_index("x")
        my_y = lax.axis_index("y")
        nbr = (my_x, 1 - my_y)

        recvbuf[...] = jnp.zeros((H, N), jnp.bfloat16)

        barrier_sem = pltpu.get_barrier_semaphore()
        pl.semaphore_signal(
            barrier_sem, inc=1, device_id=nbr,
            device_id_type=pl.DeviceIdType.MESH,
        )
        pl.semaphore_wait(barrier_sem, 1)

        rdma_d = pltpu.make_async_remote_copy(
            src_ref=dest_ref,
            dst_ref=destpeer,
            send_sem=sems_d.at[0],
            recv_sem=sems_d.at[1],
            device_id=nbr,
            device_id_type=pl.DeviceIdType.MESH,
        )
        rdma_d.start()

        lane = lax.broadcasted_iota(jnp.int32, (1, H), 1)
        slot = lax.broadcasted_iota(jnp.int32, (H, H), 0)

        mask_out = (dest_ref[...] != my_y).astype(jnp.int32)
        cs = mask_out
        for k in (1, 2, 4, 8, 16, 32, 64, 128, 256):
            cs = cs + jnp.where(lane >= k, pltpu.roll(cs, k, axis=1), 0)
        r_out = cs - mask_out
        r_loc = lane - r_out
        c_send = jnp.sum(mask_out)

        x16 = x_ref[...].astype(jnp.bfloat16)

        send_sel = jnp.where(mask_out > 0, r_out, -1)
        perm_send = (slot == send_sel).astype(jnp.bfloat16)
        sendbuf[...] = jnp.dot(
            perm_send, x16, preferred_element_type=jnp.float32
        ).astype(jnp.bfloat16)

        def chunk_rdma(k):
            return pltpu.make_async_remote_copy(
                src_ref=sendbuf.at[pl.ds(k * CH, CH)],
                dst_ref=recvbuf.at[pl.ds(k * CH, CH)],
                send_sem=ssems.at[k],
                recv_sem=rsems.at[k],
                device_id=nbr,
                device_id_type=pl.DeviceIdType.MESH,
            )

        for k in range(NCH):
            @pl.when(c_send > k * CH)
            def _(k=k):
                chunk_rdma(k).start()

        rdma_d.wait_recv()
        c_recv = jnp.sum((destpeer[...] == my_y).astype(jnp.int32))
        c_loc = H - c_send
        off_loc = jnp.where(my_y == 0, 0, c_recv)
        off_recv = jnp.where(my_y == 0, c_loc, 0)

        loc_sel = jnp.where(mask_out == 0, r_loc + off_loc, -1)
        perm_loc = (slot == loc_sel).astype(jnp.bfloat16)
        acc = jnp.dot(perm_loc, x16, preferred_element_type=jnp.float32)

        recv_sel = jnp.where(lane < c_recv, lane + off_recv, -1)
        perm_recv = (slot == recv_sel).astype(jnp.bfloat16)

        for k in range(NCH):
            @pl.when(c_recv > k * CH)
            def _(k=k):
                chunk_rdma(k).wait_recv()

        out_ref[...] = acc + jnp.dot(
            perm_recv, recvbuf[...], preferred_element_type=jnp.float32
        )

        rdma_d.wait_send()
        for k in range(NCH):
            @pl.when(c_send > k * CH)
            def _(k=k):
                chunk_rdma(k).wait_send()

    return pl.pallas_call(
        body,
        out_shape=jax.ShapeDtypeStruct((H, N), jnp.float32),
        in_specs=[
            pl.BlockSpec(memory_space=pltpu.VMEM),
            pl.BlockSpec(memory_space=pltpu.VMEM),
        ],
        out_specs=pl.BlockSpec(memory_space=pltpu.VMEM),
        scratch_shapes=[
            pltpu.VMEM((H, N), jnp.bfloat16),
            pltpu.VMEM((H, N), jnp.bfloat16),
            pltpu.VMEM((1, H), jnp.int32),
            pltpu.SemaphoreType.DMA((2,)),
            pltpu.SemaphoreType.DMA((NCH,)),
            pltpu.SemaphoreType.DMA((NCH,)),
        ],
        compiler_params=pltpu.CompilerParams(collective_id=0),
    )(x, dest_row)


# device time: 9255 ns/iter; 1.0002x vs baseline; 1.0002x over previous
import jax
import jax.numpy as jnp
from jax import lax
from jax.experimental import pallas as pl
from jax.experimental.pallas import tpu as pltpu

H = 512
N = 256
CH = 64
NCH = H // CH


def kernel(x, dest):
    dest_row = dest.reshape(1, H)

    def body(
        x_ref, dest_ref, out_ref,
        sendbuf, recvbuf, destpeer, sems_d, ssems, rsems,
    ):
        my_x = lax.axis_index("x")
        my_y = lax.axis_index("y")
        nbr = (my_x, 1 - my_y)

        recvbuf[...] = jnp.zeros((H, N), jnp.bfloat16)

        barrier_sem = pltpu.get_barrier_semaphore()
        pl.semaphore_signal(
            barrier_sem, inc=1, device_id=nbr,
            device_id_type=pl.DeviceIdType.MESH,
        )

        lane = lax.broadcasted_iota(jnp.int32, (1, H), 1)
        slot = lax.broadcasted_iota(jnp.int32, (H, H), 0)

        mask_out = (dest_ref[...] != my_y).astype(jnp.int32)
        cs = mask_out
        for k in (1, 2, 4, 8, 16, 32, 64, 128, 256):
            cs = cs + jnp.where(lane >= k, pltpu.roll(cs, k, axis=1), 0)
        r_out = cs - mask_out
        r_loc = lane - r_out
        c_send = jnp.sum(mask_out)

        x16 = x_ref[...].astype(jnp.bfloat16)

        send_sel = jnp.where(mask_out > 0, r_out, -1)
        perm_send = (slot == send_sel).astype(jnp.bfloat16)
        sendbuf[...] = jnp.dot(
            perm_send, x16, preferred_element_type=jnp.float32
        ).astype(jnp.bfloat16)

        pl.semaphore_wait(barrier_sem, 1)

        rdma_d = pltpu.make_async_remote_copy(
            src_ref=dest_ref,
            dst_ref=destpeer,
            send_sem=sems_d.at[0],
            recv_sem=sems_d.at[1],
            device_id=nbr,
            device_id_type=pl.DeviceIdType.MESH,
        )
        rdma_d.start()

        def chunk_rdma(k):
            return pltpu.make_async_remote_copy(
                src_ref=sendbuf.at[pl.ds(k * CH, CH)],
                dst_ref=recvbuf.at[pl.ds(k * CH, CH)],
                send_sem=ssems.at[k],
                recv_sem=rsems.at[k],
                device_id=nbr,
                device_id_type=pl.DeviceIdType.MESH,
            )

        for k in range(NCH):
            @pl.when(c_send > k * CH)
            def _(k=k):
                chunk_rdma(k).start()

        rdma_d.wait_recv()
        c_recv = jnp.sum((destpeer[...] == my_y).astype(jnp.int32))
        c_loc = H - c_send
        off_loc = jnp.where(my_y == 0, 0, c_recv)
        off_recv = jnp.where(my_y == 0, c_loc, 0)

        loc_sel = jnp.where(mask_out == 0, r_loc + off_loc, -1)
        perm_loc = (slot == loc_sel).astype(jnp.bfloat16)
        acc = jnp.dot(perm_loc, x16, preferred_element_type=jnp.float32)

        recv_sel = jnp.where(lane < c_recv, lane + off_recv, -1)
        perm_recv = (slot == recv_sel).astype(jnp.bfloat16)

        for k in range(NCH):
            @pl.when(c_recv > k * CH)
            def _(k=k):
                chunk_rdma(k).wait_recv()

        out_ref[...] = acc + jnp.dot(
            perm_recv, recvbuf[...], preferred_element_type=jnp.float32
        )

        rdma_d.wait_send()
        for k in range(NCH):
            @pl.when(c_send > k * CH)
            def _(k=k):
                chunk_rdma(k).wait_send()

    return pl.pallas_call(
        body,
        out_shape=jax.ShapeDtypeStruct((H, N), jnp.float32),
        in_specs=[
            pl.BlockSpec(memory_space=pltpu.VMEM),
            pl.BlockSpec(memory_space=pltpu.VMEM),
        ],
        out_specs=pl.BlockSpec(memory_space=pltpu.VMEM),
        scratch_shapes=[
            pltpu.VMEM((H, N), jnp.bfloat16),
            pltpu.VMEM((H, N), jnp.bfloat16),
            pltpu.VMEM((1, H), jnp.int32),
            pltpu.SemaphoreType.DMA((2,)),
            pltpu.SemaphoreType.DMA((NCH,)),
            pltpu.SemaphoreType.DMA((NCH,)),
        ],
        compiler_params=pltpu.CompilerParams(collective_id=0),
    )(x, dest_row)
